# baseline (speedup 1.0000x reference)
import functools

import jax
import jax.numpy as jnp
from jax import lax
from jax.experimental import pallas as pl
from jax.experimental.pallas import tpu as pltpu
from jax.experimental.pallas import tpu_sc as plsc

N = 10000
NP = 10240
E = 320000
EP = 327680
D = 128
H = 16
NC = 2
NS = 16
NW = NC * NS
EPW = EP // NW
MB = 2560
MCHUNK = EPW // MB
EPS = EP // NS
DB = 2048
DCHUNK = EPS // DB
NPER = NP // NS

_mesh = plsc.VectorSubcoreMesh(core_axis_name="c", subcore_axis_name="s")
_sc_params = pltpu.CompilerParams(use_tc_tiling_on_sc=False)


@functools.partial(
    pl.kernel,
    out_type=jax.ShapeDtypeStruct((NC, NP), jnp.float32),
    mesh=_mesh,
    compiler_params=_sc_params,
    scratch_types=[
        pltpu.VMEM_SHARED((NP,), jnp.float32),
        pltpu.VMEM((DB,), jnp.int32),
        pltpu.VMEM((DB,), jnp.int32),
        pltpu.VMEM((DB,), jnp.float32),
        pltpu.VMEM((NPER,), jnp.float32),
        pltpu.SemaphoreType.DMA,
        pltpu.SemaphoreType.DMA,
    ],
)
def _sc_degrees(src_hbm, dst_hbm, deg_hbm,
                deg_sh, idx0_v, idx1_v, ones_v, z_v, sem0, sem1):
    c = lax.axis_index("c")
    s = lax.axis_index("s")

    @pl.loop(0, DB // 16)
    def _(j):
        ones_v[pl.ds(j * 16, 16)] = jnp.ones((16,), jnp.float32)

    @pl.loop(0, NPER // 16)
    def _(j):
        z_v[pl.ds(j * 16, 16)] = jnp.zeros((16,), jnp.float32)

    pltpu.sync_copy(z_v, deg_sh.at[pl.ds(s * NPER, NPER)])
    plsc.subcore_barrier()

    def run(e_hbm):
        idx = (idx0_v, idx1_v)
        sem = (sem0, sem1)
        pend = [None, None]
        for k in range(DCHUNK):
            b = k % 2
            if pend[b] is not None:
                pend[b].wait()
            pltpu.sync_copy(e_hbm.at[pl.ds(s * EPS + k * DB, DB)], idx[b])
            pend[b] = pltpu.async_copy(ones_v, deg_sh.at[idx[b]], sem[b],
                                       add=True)
        pend[0].wait()
        pend[1].wait()

    @pl.when(c == 0)
    def _():
        run(src_hbm)

    @pl.when(c == 1)
    def _():
        run(dst_hbm)

    plsc.subcore_barrier()
    pltpu.sync_copy(deg_sh.at[pl.ds(s * NPER, NPER)],
                    deg_hbm.at[c, pl.ds(s * NPER, NPER)])


@functools.partial(
    pl.kernel,
    out_type=jax.ShapeDtypeStruct((NC, NP, H), jnp.float32),
    mesh=_mesh,
    compiler_params=_sc_params,
    scratch_types=[
        pltpu.VMEM_SHARED((NP, H), jnp.float32),
        [pltpu.VMEM((MB,), jnp.int32) for _ in range(MCHUNK)],
        [pltpu.VMEM((MB,), jnp.int32) for _ in range(MCHUNK)],
        pltpu.VMEM((MB, H), jnp.float32),
        pltpu.VMEM((MB, H), jnp.float32),
        pltpu.SemaphoreType.DMA,
        pltpu.SemaphoreType.DMA,
        pltpu.SemaphoreType.DMA,
        pltpu.SemaphoreType.DMA,
    ],
)
def _sc_messages(h_hbm, src_hbm, dst_hbm, aggp_hbm,
                 agg_sh, sidx, didx, rows0_v, rows1_v,
                 semg0, semg1, sems0, sems1):
    c = lax.axis_index("c")
    s = lax.axis_index("s")

    @pl.loop(0, NPER)
    def _(j):
        rows0_v[j, :] = jnp.zeros((H,), jnp.float32)

    pltpu.sync_copy(rows0_v.at[pl.ds(0, NPER)], agg_sh.at[pl.ds(s * NPER, NPER)])
    e0 = (c * NS + s) * EPW
    for k in range(MCHUNK):
        pltpu.sync_copy(src_hbm.at[pl.ds(e0 + k * MB, MB)], sidx[k])
        pltpu.sync_copy(dst_hbm.at[pl.ds(e0 + k * MB, MB)], didx[k])
    plsc.subcore_barrier()

    rows = (rows0_v, rows1_v)
    semg = (semg0, semg1)
    sems = (sems0, sems1)
    gat = [None] * MCHUNK
    sca = [None] * MCHUNK
    for k in range(MCHUNK):
        b = k % 2
        if k >= 2:
            sca[k - 2].wait()
        gat[k] = pltpu.async_copy(h_hbm.at[sidx[k]], rows[b], semg[b])
        if k >= 1:
            bp = (k - 1) % 2
            gat[k - 1].wait()
            sca[k - 1] = pltpu.async_copy(rows[bp], agg_sh.at[didx[k - 1]],
                                          sems[bp], add=True)
    gat[MCHUNK - 1].wait()
    bl = (MCHUNK - 1) % 2
    sca[MCHUNK - 1] = pltpu.async_copy(rows[bl], agg_sh.at[didx[MCHUNK - 1]],
                                       sems[bl], add=True)
    sca[MCHUNK - 2].wait()
    sca[MCHUNK - 1].wait()

    plsc.subcore_barrier()
    pltpu.sync_copy(agg_sh.at[pl.ds(s * NPER, NPER)],
                    aggp_hbm.at[c, pl.ds(s * NPER, NPER)])


def _tc_prep1_body(f_ref, w1_ref, deg_ref, h1s_ref, no_ref, ni_ref):
    no = lax.rsqrt(jnp.maximum(deg_ref[0], 1.0)).reshape(NP, 1)
    ni = lax.rsqrt(jnp.maximum(deg_ref[1], 1.0)).reshape(NP, 1)
    h = jnp.dot(f_ref[...], w1_ref[...], preferred_element_type=jnp.float32)
    h1s_ref[...] = h * no
    no_ref[...] = no
    ni_ref[...] = ni


_tc_prep1 = pl.pallas_call(
    _tc_prep1_body,
    out_shape=[
        jax.ShapeDtypeStruct((NP, H), jnp.float32),
        jax.ShapeDtypeStruct((NP, 1), jnp.float32),
        jax.ShapeDtypeStruct((NP, 1), jnp.float32),
    ],
)


def _tc_mid_body(aggp_ref, ni_ref, b1_ref, w2_ref, no_ref, h2s_ref):
    agg = aggp_ref[0] + aggp_ref[1]
    y = jnp.maximum(agg * ni_ref[...] + b1_ref[...], 0.0)
    h2 = jnp.dot(y, w2_ref[...], preferred_element_type=jnp.float32)
    h2s_ref[...] = h2 * no_ref[...]


_tc_mid = pl.pallas_call(
    _tc_mid_body,
    out_shape=jax.ShapeDtypeStruct((NP, H), jnp.float32),
)


def _tc_final_body(aggp_ref, ni_ref, b2_ref, out_ref):
    agg = aggp_ref[0] + aggp_ref[1]
    out_ref[...] = agg * ni_ref[...] + b2_ref[...]


_tc_final = pl.pallas_call(
    _tc_final_body,
    out_shape=jax.ShapeDtypeStruct((NP, H), jnp.float32),
)


def kernel(features, edge_index, W1, b1, W2, b2):
    src = edge_index[0]
    dst = edge_index[1]
    pad = jnp.full((EP - E,), N, jnp.int32)
    src_p = jnp.concatenate([src, pad])
    dst_p = jnp.concatenate([dst, pad])
    f_p = jnp.zeros((NP, D), jnp.float32).at[:N].set(features)

    deg = _sc_degrees(src_p, dst_p)
    h1s, no, ni = _tc_prep1(f_p, W1, deg)
    agg1p = _sc_messages(h1s, src_p, dst_p)
    h2s = _tc_mid(agg1p, ni, b1.reshape(1, H), W2, no)
    agg2p = _sc_messages(h2s, src_p, dst_p)
    out_p = _tc_final(agg2p, ni, b2.reshape(1, H))
    return out_p[:N]


# speedup vs baseline: 1.2724x; 1.2724x over previous
import functools

import jax
import jax.numpy as jnp
from jax import lax
from jax.experimental import pallas as pl
from jax.experimental.pallas import tpu as pltpu
from jax.experimental.pallas import tpu_sc as plsc

N = 10000
NP = 10240
E = 320000
EP = 327680
D = 128
H = 16
NC = 2
NS = 16
NW = NC * NS
EPW = EP // NW
MB = 2560
MCHUNK = EPW // MB
EPS = EP // NS
NR = NP // 16
NPER = NP // NS

_mesh = plsc.VectorSubcoreMesh(core_axis_name="c", subcore_axis_name="s")
_sc_params = pltpu.CompilerParams(use_tc_tiling_on_sc=False)
_sc_params_nl = pltpu.CompilerParams(use_tc_tiling_on_sc=False,
                                     needs_layout_passes=False)


@functools.partial(
    pl.kernel,
    out_type=jax.ShapeDtypeStruct((NC, NR, 16), jnp.float32),
    mesh=_mesh,
    compiler_params=_sc_params_nl,
    scratch_types=[
        pltpu.VMEM_SHARED((NR, 16), jnp.float32),
        pltpu.VMEM((NR, 16), jnp.float32),
        pltpu.VMEM((EPS,), jnp.int32),
        pltpu.VMEM((NR,), jnp.int32),
    ],
)
def _sc_degrees(edges_hbm, deg_hbm, deg_sh, hist_v, idx_v, ident_v):
    c = lax.axis_index("c")
    s = lax.axis_index("s")

    @pl.loop(0, NR)
    def _(j):
        hist_v[j, :] = jnp.zeros((16,), jnp.float32)

    @pl.loop(0, NR // 16)
    def _(j):
        ident_v[pl.ds(j * 16, 16)] = lax.iota(jnp.int32, 16) + j * 16

    pltpu.sync_copy(hist_v.at[pl.ds(0, NR // NS)],
                    deg_sh.at[pl.ds(s * (NR // NS), NR // NS)])
    pltpu.sync_copy(edges_hbm.at[c, pl.ds(s * EPS, EPS)], idx_v)
    plsc.subcore_barrier()

    ones16 = jnp.ones((16,), jnp.float32)

    @pl.loop(0, EPS // 16)
    def _(j):
        v = idx_v[pl.ds(j * 16, 16)]
        row = lax.shift_right_logical(v, 4)
        col = lax.bitwise_and(v, 15)
        plsc.addupdate_scatter(hist_v, [row, col], ones16)

    pltpu.sync_copy(hist_v, deg_sh.at[ident_v], add=True)
    plsc.subcore_barrier()
    pltpu.sync_copy(deg_sh.at[pl.ds(s * (NR // NS), NR // NS)],
                    deg_hbm.at[c, pl.ds(s * (NR // NS), NR // NS)])


@functools.partial(
    pl.kernel,
    out_type=jax.ShapeDtypeStruct((NC, NP, H), jnp.float32),
    mesh=_mesh,
    compiler_params=_sc_params,
    scratch_types=[
        pltpu.VMEM_SHARED((NP, H), jnp.float32),
        pltpu.VMEM_SHARED((NP, H), jnp.float32),
        [pltpu.VMEM((MB,), jnp.int32) for _ in range(MCHUNK)],
        [pltpu.VMEM((MB,), jnp.int32) for _ in range(MCHUNK)],
        pltpu.VMEM((MB, H), jnp.float32),
        pltpu.VMEM((MB, H), jnp.float32),
        pltpu.SemaphoreType.DMA,
        pltpu.SemaphoreType.DMA,
        pltpu.SemaphoreType.DMA,
        pltpu.SemaphoreType.DMA,
    ],
)
def _sc_messages(h_hbm, src_hbm, dst_hbm, aggp_hbm,
                 h_sh, agg_sh, sidx, didx, rows0_v, rows1_v,
                 semg0, semg1, sems0, sems1):
    c = lax.axis_index("c")
    s = lax.axis_index("s")

    @pl.loop(0, NPER)
    def _(j):
        rows0_v[j, :] = jnp.zeros((H,), jnp.float32)

    pltpu.sync_copy(h_hbm.at[pl.ds(s * NPER, NPER)],
                    h_sh.at[pl.ds(s * NPER, NPER)])
    pltpu.sync_copy(rows0_v.at[pl.ds(0, NPER)], agg_sh.at[pl.ds(s * NPER, NPER)])
    e0 = (c * NS + s) * EPW
    for k in range(MCHUNK):
        pltpu.sync_copy(src_hbm.at[pl.ds(e0 + k * MB, MB)], sidx[k])
        pltpu.sync_copy(dst_hbm.at[pl.ds(e0 + k * MB, MB)], didx[k])
    plsc.subcore_barrier()

    rows = (rows0_v, rows1_v)
    semg = (semg0, semg1)
    sems = (sems0, sems1)
    gat = [None] * MCHUNK
    sca = [None] * MCHUNK
    for k in range(MCHUNK):
        b = k % 2
        if k >= 2:
            sca[k - 2].wait()
        gat[k] = pltpu.async_copy(h_sh.at[sidx[k]], rows[b], semg[b])
        if k >= 1:
            bp = (k - 1) % 2
            gat[k - 1].wait()
            sca[k - 1] = pltpu.async_copy(rows[bp], agg_sh.at[didx[k - 1]],
                                          sems[bp], add=True)
    gat[MCHUNK - 1].wait()
    bl = (MCHUNK - 1) % 2
    sca[MCHUNK - 1] = pltpu.async_copy(rows[bl], agg_sh.at[didx[MCHUNK - 1]],
                                       sems[bl], add=True)
    sca[MCHUNK - 2].wait()
    sca[MCHUNK - 1].wait()

    plsc.subcore_barrier()
    pltpu.sync_copy(agg_sh.at[pl.ds(s * NPER, NPER)],
                    aggp_hbm.at[c, pl.ds(s * NPER, NPER)])


def _tc_prep1_body(f_ref, w1_ref, deg_ref, h1s_ref, no_ref, ni_ref):
    no = lax.rsqrt(jnp.maximum(deg_ref[0], 1.0)).reshape(NP, 1)
    ni = lax.rsqrt(jnp.maximum(deg_ref[1], 1.0)).reshape(NP, 1)
    h = jnp.dot(f_ref[...], w1_ref[...], preferred_element_type=jnp.float32)
    h1s_ref[...] = h * no
    no_ref[...] = no
    ni_ref[...] = ni


_tc_prep1 = pl.pallas_call(
    _tc_prep1_body,
    out_shape=[
        jax.ShapeDtypeStruct((NP, H), jnp.float32),
        jax.ShapeDtypeStruct((NP, 1), jnp.float32),
        jax.ShapeDtypeStruct((NP, 1), jnp.float32),
    ],
)


def _tc_mid_body(aggp_ref, ni_ref, b1_ref, w2_ref, no_ref, h2s_ref):
    agg = aggp_ref[0] + aggp_ref[1]
    y = jnp.maximum(agg * ni_ref[...] + b1_ref[...], 0.0)
    h2 = jnp.dot(y, w2_ref[...], preferred_element_type=jnp.float32)
    h2s_ref[...] = h2 * no_ref[...]


_tc_mid = pl.pallas_call(
    _tc_mid_body,
    out_shape=jax.ShapeDtypeStruct((NP, H), jnp.float32),
)


def _tc_final_body(aggp_ref, ni_ref, b2_ref, out_ref):
    agg = aggp_ref[0] + aggp_ref[1]
    out_ref[...] = agg * ni_ref[...] + b2_ref[...]


_tc_final = pl.pallas_call(
    _tc_final_body,
    out_shape=jax.ShapeDtypeStruct((NP, H), jnp.float32),
)


def kernel(features, edge_index, W1, b1, W2, b2):
    src = edge_index[0]
    dst = edge_index[1]
    pad = jnp.full((EP - E,), N, jnp.int32)
    src_p = jnp.concatenate([src, pad])
    dst_p = jnp.concatenate([dst, pad])
    f_p = jnp.zeros((NP, D), jnp.float32).at[:N].set(features)

    deg = _sc_degrees(jnp.stack([src_p, dst_p])).reshape(NC, NP)
    h1s, no, ni = _tc_prep1(f_p, W1, deg)
    agg1p = _sc_messages(h1s, src_p, dst_p)
    h2s = _tc_mid(agg1p, ni, b1.reshape(1, H), W2, no)
    agg2p = _sc_messages(h2s, src_p, dst_p)
    out_p = _tc_final(agg2p, ni, b2.reshape(1, H))
    return out_p[:N]


# speedup vs baseline: 1.6255x; 1.2775x over previous
import functools

import jax
import jax.numpy as jnp
from jax import lax
from jax.experimental import pallas as pl
from jax.experimental.pallas import tpu as pltpu
from jax.experimental.pallas import tpu_sc as plsc

N = 10000
NP = 10240
E = 320000
D = 128
H = 16
NC = 2
NS = 16
NW = NC * NS
EPW = E // NW
MB = 2000
MCHUNK = EPW // MB
EPS = E // NS
NR = NP // 16
NPER = NP // NS

_mesh = plsc.VectorSubcoreMesh(core_axis_name="c", subcore_axis_name="s")
_sc_params = pltpu.CompilerParams(use_tc_tiling_on_sc=False)
_sc_params_nl = pltpu.CompilerParams(use_tc_tiling_on_sc=False,
                                     needs_layout_passes=False)


@functools.partial(
    pl.kernel,
    out_type=jax.ShapeDtypeStruct((NC, NR, 16), jnp.float32),
    mesh=_mesh,
    compiler_params=_sc_params_nl,
    scratch_types=[
        pltpu.VMEM_SHARED((NR, 16), jnp.float32),
        pltpu.VMEM((NR, 16), jnp.float32),
        pltpu.VMEM((EPS,), jnp.int32),
        pltpu.VMEM((NR,), jnp.int32),
    ],
)
def _sc_degrees(edges_hbm, deg_hbm, deg_sh, hist_v, idx_v, ident_v):
    c = lax.axis_index("c")
    s = lax.axis_index("s")

    @pl.loop(0, NR)
    def _(j):
        hist_v[j, :] = jnp.zeros((16,), jnp.float32)

    @pl.loop(0, NR // 16)
    def _(j):
        ident_v[pl.ds(j * 16, 16)] = lax.iota(jnp.int32, 16) + j * 16

    pltpu.sync_copy(hist_v.at[pl.ds(0, NR // NS)],
                    deg_sh.at[pl.ds(s * (NR // NS), NR // NS)])
    pltpu.sync_copy(edges_hbm.at[c, pl.ds(s * EPS, EPS)], idx_v)
    plsc.subcore_barrier()

    ones16 = jnp.ones((16,), jnp.float32)

    @pl.loop(0, EPS // 16)
    def _(j):
        v = idx_v[pl.ds(j * 16, 16)]
        row = lax.shift_right_logical(v, 4)
        col = lax.bitwise_and(v, 15)
        plsc.addupdate_scatter(hist_v, [row, col], ones16)

    pltpu.sync_copy(hist_v, deg_sh.at[ident_v], add=True)
    plsc.subcore_barrier()
    pltpu.sync_copy(deg_sh.at[pl.ds(s * (NR // NS), NR // NS)],
                    deg_hbm.at[c, pl.ds(s * (NR // NS), NR // NS)])


@functools.partial(
    pl.kernel,
    out_type=jax.ShapeDtypeStruct((NC, NP, H), jnp.float32),
    mesh=_mesh,
    compiler_params=_sc_params,
    scratch_types=[
        pltpu.VMEM_SHARED((NP, H), jnp.float32),
        pltpu.VMEM_SHARED((NP, H), jnp.float32),
        [pltpu.VMEM((MB,), jnp.int32) for _ in range(MCHUNK)],
        [pltpu.VMEM((MB,), jnp.int32) for _ in range(MCHUNK)],
        pltpu.VMEM((MB, H), jnp.float32),
        pltpu.VMEM((MB, H), jnp.float32),
        pltpu.SemaphoreType.DMA,
        pltpu.SemaphoreType.DMA,
        pltpu.SemaphoreType.DMA,
        pltpu.SemaphoreType.DMA,
    ],
)
def _sc_messages(h_hbm, edges_hbm, aggp_hbm,
                 h_sh, agg_sh, sidx, didx, rows0_v, rows1_v,
                 semg0, semg1, sems0, sems1):
    c = lax.axis_index("c")
    s = lax.axis_index("s")

    @pl.loop(0, NPER)
    def _(j):
        rows0_v[j, :] = jnp.zeros((H,), jnp.float32)

    pltpu.sync_copy(h_hbm.at[pl.ds(s * NPER, NPER)],
                    h_sh.at[pl.ds(s * NPER, NPER)])
    pltpu.sync_copy(rows0_v.at[pl.ds(0, NPER)],
                    agg_sh.at[pl.ds(s * NPER, NPER)])
    e0 = (c * NS + s) * EPW
    for k in range(MCHUNK):
        pltpu.sync_copy(edges_hbm.at[0, pl.ds(e0 + k * MB, MB)], sidx[k])
        pltpu.sync_copy(edges_hbm.at[1, pl.ds(e0 + k * MB, MB)], didx[k])
    plsc.subcore_barrier()

    rows = (rows0_v, rows1_v)
    semg = (semg0, semg1)
    sems = (sems0, sems1)
    gat = [None] * MCHUNK
    sca = [None] * MCHUNK
    for k in range(MCHUNK):
        b = k % 2
        if k >= 2:
            sca[k - 2].wait()
        gat[k] = pltpu.async_copy(h_sh.at[sidx[k]], rows[b], semg[b])
        if k >= 1:
            bp = (k - 1) % 2
            gat[k - 1].wait()
            sca[k - 1] = pltpu.async_copy(rows[bp], agg_sh.at[didx[k - 1]],
                                          sems[bp], add=True)
    gat[MCHUNK - 1].wait()
    bl = (MCHUNK - 1) % 2
    sca[MCHUNK - 1] = pltpu.async_copy(rows[bl], agg_sh.at[didx[MCHUNK - 1]],
                                       sems[bl], add=True)
    sca[MCHUNK - 2].wait()
    sca[MCHUNK - 1].wait()

    plsc.subcore_barrier()
    pltpu.sync_copy(agg_sh.at[pl.ds(s * NPER, NPER)],
                    aggp_hbm.at[c, pl.ds(s * NPER, NPER)])


def _tc_prep1_body(f_ref, w1_ref, deg_ref, h1s_ref, no_ref, ni_ref):
    no = lax.rsqrt(jnp.maximum(deg_ref[0], 1.0)).reshape(NP, 1)
    ni = lax.rsqrt(jnp.maximum(deg_ref[1], 1.0)).reshape(NP, 1)
    h = jnp.dot(f_ref[...], w1_ref[...], preferred_element_type=jnp.float32)
    h1s_ref[...] = jnp.concatenate(
        [h * no[:N], jnp.zeros((NP - N, H), jnp.float32)], axis=0)
    no_ref[...] = no
    ni_ref[...] = ni


_tc_prep1 = pl.pallas_call(
    _tc_prep1_body,
    out_shape=[
        jax.ShapeDtypeStruct((NP, H), jnp.float32),
        jax.ShapeDtypeStruct((NP, 1), jnp.float32),
        jax.ShapeDtypeStruct((NP, 1), jnp.float32),
    ],
)


def _tc_mid_body(aggp_ref, ni_ref, b1_ref, w2_ref, no_ref, h2s_ref):
    agg = aggp_ref[0] + aggp_ref[1]
    y = jnp.maximum(agg * ni_ref[...] + b1_ref[...], 0.0)
    h2 = jnp.dot(y, w2_ref[...], preferred_element_type=jnp.float32)
    h2s_ref[...] = h2 * no_ref[...]


_tc_mid = pl.pallas_call(
    _tc_mid_body,
    out_shape=jax.ShapeDtypeStruct((NP, H), jnp.float32),
)


def _tc_final_body(aggp_ref, ni_ref, b2_ref, out_ref):
    agg = aggp_ref[0, :N, :] + aggp_ref[1, :N, :]
    out_ref[...] = agg * ni_ref[:N, :] + b2_ref[...]


_tc_final = pl.pallas_call(
    _tc_final_body,
    out_shape=jax.ShapeDtypeStruct((N, H), jnp.float32),
)


def kernel(features, edge_index, W1, b1, W2, b2):
    deg = _sc_degrees(edge_index).reshape(NC, NP)
    h1s, no, ni = _tc_prep1(features, W1, deg)
    agg1p = _sc_messages(h1s, edge_index)
    h2s = _tc_mid(agg1p, ni, b1.reshape(1, H), W2, no)
    agg2p = _sc_messages(h2s, edge_index)
    return _tc_final(agg2p, ni, b2.reshape(1, H))
